# baseline (speedup 1.0000x reference)
import functools

import jax
import jax.numpy as jnp
from jax.experimental import pallas as pl

HN = 64
PREF = 16
SCAN = 11
STATE = 6
OBS = SCAN + STATE + PREF
ENC = 128
REPR = ENC + PREF
K = 6
THR = 4.0
NL = 3
SB = 8
BIG = 1e30

_F32 = jnp.float32


def _silu(v):
    return v / (1.0 + jnp.exp(-v))


_BF16 = jnp.bfloat16


def _dot(a, b):
    return jnp.dot(a.astype(_BF16), b.astype(_BF16), preferred_element_type=_F32)


def _body(obs_ref, ped_ref, pedt_ref, stt_ref,
          ew1, eb1, ew2, eb2,
          few1, feb1, few2, feb2,
          fxw1, fxb1, fxw2c, fxb2,
          fhw1, fhb1, fhw2, fhb2,
          out_ref):
    n = SB * HN
    obs = obs_ref[...]
    ped = ped_ref[...]

    ei = obs[:, :STATE + SCAN]
    h1 = jnp.maximum(_dot(ei, ew1[...]) + eb1[...], 0.0)
    henc = _dot(h1, ew2[...]) + eb2[...]
    h = jnp.concatenate([henc, obs[:, STATE + SCAN:OBS]], axis=1)

    stcols = obs[:, :STATE]
    mcol = jnp.max(jnp.where(stcols != 0.0, 1.0, 0.0), axis=1, keepdims=True)
    stt = stt_ref[...]
    mrow3 = jnp.max(jnp.where(stt != 0.0, 1.0, 0.0), axis=1, keepdims=True)
    mrow = jnp.broadcast_to(mrow3, (SB, HN, HN)).reshape(n, HN)

    pedt = pedt_ref[...]
    pxr = jnp.broadcast_to(pedt[:, 0:1, :], (SB, HN, HN)).reshape(n, HN)
    pyr = jnp.broadcast_to(pedt[:, 1:2, :], (SB, HN, HN)).reshape(n, HN)
    dx = pxr - ped[:, 0:1]
    dy = pyr - ped[:, 1:2]
    d2 = dx * dx + dy * dy
    dist = jnp.sqrt(d2)
    valid = (mcol * mrow) > 0.0
    dist = jnp.where(valid, dist, BIG)

    iota = jax.lax.broadcasted_iota(jnp.int32, (n, HN), 1)
    rowid = jax.lax.broadcasted_iota(jnp.int32, (n, 1), 0)
    goff = (rowid // HN) * HN
    giota = jax.lax.broadcasted_iota(jnp.int32, (n, n), 1)
    cur = dist
    w_parts = []
    mb_parts = []
    for _ in range(K):
        mval = jnp.min(cur, axis=1, keepdims=True)
        ismin = cur == mval
        sel = jnp.min(jnp.where(ismin, iota, HN), axis=1, keepdims=True)
        onehot_local = iota == sel
        gsel = sel + goff
        w_parts.append(jnp.where(giota == gsel, 1.0, 0.0))
        mb_parts.append(jnp.where(mval <= THR, 1.0, 0.0))
        cur = jnp.where(onehot_local, BIG, cur)

    W = jnp.concatenate(w_parts, axis=0)
    mbf = jnp.concatenate(mb_parts, axis=0)
    Wm = (W * mbf).astype(_BF16)
    neigh_num = mb_parts[0]
    for k in range(1, K):
        neigh_num = neigh_num + mb_parts[k]
    inv_nn = 1.0 / (neigh_num + 1e-6)

    for i in range(NL):
        fe1 = few1[i]
        A = _dot(h, fe1[0:REPR]) + feb1[i]
        B = _dot(h, fe1[REPR:2 * REPR])
        c = fe1[2 * REPR:2 * REPR + 1]

        hb = _dot(Wm, B)
        Arep = jnp.concatenate([A] * K, axis=0)
        pg = _dot(Wm, ped)
        pedi = jnp.concatenate([ped] * K, axis=0)
        rel = pg - mbf * pedi
        sq = jnp.sum(rel * rel, axis=1, keepdims=True)
        dists = jnp.sqrt(sq)

        pre1 = Arep + hb + dists * c
        u = _silu(pre1)
        mij = _silu(_dot(u, few2[i]) + feb2[i]) * mbf

        fxh = _silu(_dot(mij, fxw1[i]) + fxb1[i])
        fx = _dot(fxh, fxw2c[i]) + fxb2[i]

        agg = rel * fx
        aggs = agg[0:n]
        mi = mij[0:n]
        for k in range(1, K):
            aggs = aggs + agg[k * n:(k + 1) * n]
            mi = mi + mij[k * n:(k + 1) * n]
        ped = ped + inv_nn * aggs

        fh1 = fhw1[i]
        pre = _dot(h, fh1[0:REPR]) + _dot(mi, fh1[REPR:2 * REPR]) + fhb1[i]
        h = h + _dot(_silu(pre), fhw2[i]) + fhb2[i]

    out_ref[...] = h


@functools.partial(jax.jit, static_argnames=())
def kernel(x, enc_w1, enc_b1, enc_w2, enc_b2, fe_w1, fe_b1, fe_w2, fe_b2,
           fx_w1, fx_b1, fx_w2, fx_b2, fh_w1, fh_b1, fh_w2, fh_b2):
    bs = x.shape[0]
    obs3 = x[:, :HN * OBS].reshape(bs, HN, OBS)
    ped3 = x[:, HN * OBS:].reshape(bs, HN, 2)
    obs2 = obs3.reshape(bs * HN, OBS)
    ped2 = ped3.reshape(bs * HN, 2)
    pedt = jnp.swapaxes(ped3, 1, 2)
    stt = jnp.swapaxes(obs3[..., :STATE], 1, 2)

    ew1 = enc_w1.astype(jnp.bfloat16)
    ew2 = enc_w2.astype(jnp.bfloat16)
    few1 = fe_w1.astype(jnp.bfloat16)
    few2 = fe_w2.astype(jnp.bfloat16)
    fxw1 = fx_w1.astype(jnp.bfloat16)
    fhw1 = fh_w1.astype(jnp.bfloat16)
    fhw2 = fh_w2.astype(jnp.bfloat16)
    eb1 = enc_b1.reshape(1, ENC)
    eb2 = enc_b2.reshape(1, ENC)
    feb1 = fe_b1.reshape(NL, 1, REPR)
    feb2 = fe_b2.reshape(NL, 1, REPR)
    fxb1 = fx_b1.reshape(NL, 1, REPR)
    fxw2c = fx_w2.astype(jnp.bfloat16)
    fxb2 = fx_b2.reshape(NL, 1, 1)
    fhb1 = fh_b1.reshape(NL, 1, REPR)
    fhb2 = fh_b2.reshape(NL, 1, REPR)

    n = SB * HN
    full = lambda arr: pl.BlockSpec(arr.shape, lambda b: (0,) * arr.ndim)
    out = pl.pallas_call(
        _body,
        grid=(bs // SB,),
        in_specs=[
            pl.BlockSpec((n, OBS), lambda b: (b, 0)),
            pl.BlockSpec((n, 2), lambda b: (b, 0)),
            pl.BlockSpec((SB, 2, HN), lambda b: (b, 0, 0)),
            pl.BlockSpec((SB, STATE, HN), lambda b: (b, 0, 0)),
            full(ew1), full(eb1), full(ew2), full(eb2),
            full(few1), full(feb1), full(few2), full(feb2),
            full(fxw1), full(fxb1), full(fxw2c), full(fxb2),
            full(fhw1), full(fhb1), full(fhw2), full(fhb2),
        ],
        out_specs=pl.BlockSpec((n, REPR), lambda b: (b, 0)),
        out_shape=jax.ShapeDtypeStruct((bs * HN, REPR), jnp.float32),
    )(obs2, ped2, pedt, stt,
      ew1, eb1, ew2, eb2,
      few1, feb1, few2, feb2,
      fxw1, fxb1, fxw2c, fxb2,
      fhw1, fhb1, fhw2, fhb2)
    return out


# speedup vs baseline: 1.0461x; 1.0461x over previous
import functools

import jax
import jax.numpy as jnp
from jax.experimental import pallas as pl
from jax.experimental.pallas import tpu as pltpu

HN = 64
PREF = 16
SCAN = 11
STATE = 6
OBS = SCAN + STATE + PREF
ENC = 128
REPR = ENC + PREF
K = 6
THR = 4.0
NL = 3
SB = 8
BIG = 1e30

_F32 = jnp.float32


def _silu(v):
    return v * (1.0 / (1.0 + jnp.exp(-v)))


def _dot(a, b):
    return jnp.dot(a, b, preferred_element_type=_F32)


def _body(obs_ref, ped_ref, pedt_ref, stt_ref,
          ew1, eb1, ew2, eb2,
          few1, feb1, few2, feb2,
          fxw1, fxb1, fxw2t, fxb2,
          fhw1, fhb1, fhw2, fhb2,
          out_ref):
    n = SB * HN
    obs = obs_ref[...]
    ped = ped_ref[...]

    ei = obs[:, :STATE + SCAN]
    h1 = jnp.maximum(_dot(ei, ew1[...]) + eb1[...], 0.0)
    henc = _dot(h1, ew2[...]) + eb2[...]
    h = jnp.concatenate([henc, obs[:, STATE + SCAN:OBS]], axis=1)

    stcols = obs[:, :STATE]
    mcol = jnp.max(jnp.where(stcols != 0.0, 1.0, 0.0), axis=1, keepdims=True)
    stt = stt_ref[...]
    mrow3 = jnp.max(jnp.where(stt != 0.0, 1.0, 0.0), axis=1, keepdims=True)
    mrow = jnp.broadcast_to(mrow3, (SB, HN, HN)).reshape(n, HN)

    pedt = pedt_ref[...]
    pxr = jnp.broadcast_to(pedt[:, 0:1, :], (SB, HN, HN)).reshape(n, HN)
    pyr = jnp.broadcast_to(pedt[:, 1:2, :], (SB, HN, HN)).reshape(n, HN)
    dx = pxr - ped[:, 0:1]
    dy = pyr - ped[:, 1:2]
    d2 = dx * dx + dy * dy
    dist = jnp.sqrt(d2)
    valid = (mcol * mrow) > 0.0
    dist = jnp.where(valid, dist, BIG)

    iota = jax.lax.broadcasted_iota(jnp.int32, (n, HN), 1)
    rowid = jax.lax.broadcasted_iota(jnp.int32, (n, 1), 0)
    goff = (rowid // HN) * HN
    giota = jax.lax.broadcasted_iota(jnp.int32, (n, n), 1)
    cur = dist
    w_parts = []
    mb_parts = []
    for _ in range(K):
        mval = jnp.min(cur, axis=1, keepdims=True)
        ismin = cur == mval
        sel = jnp.min(jnp.where(ismin, iota, HN), axis=1, keepdims=True)
        onehot_local = iota == sel
        gsel = sel + goff
        w_parts.append(jnp.where(giota == gsel, 1.0, 0.0))
        mb_parts.append(jnp.where(mval <= THR, 1.0, 0.0))
        cur = jnp.where(onehot_local, BIG, cur)

    W = jnp.concatenate(w_parts, axis=0)
    mbf = jnp.concatenate(mb_parts, axis=0)
    Wm = W * mbf
    neigh_num = mb_parts[0]
    for k in range(1, K):
        neigh_num = neigh_num + mb_parts[k]
    inv_nn = 1.0 / (neigh_num + 1e-6)

    for i in range(NL):
        fe1 = few1[i]
        A = _dot(h, fe1[0:REPR])
        B = _dot(h, fe1[REPR:2 * REPR])
        c = fe1[2 * REPR:2 * REPR + 1]

        hb = _dot(Wm, B)
        Arep = jnp.concatenate([A] * K, axis=0)
        pg = _dot(Wm, ped)
        pedi = jnp.concatenate([ped] * K, axis=0)
        rel = pg - mbf * pedi
        sq = jnp.sum(rel * rel, axis=1, keepdims=True)
        dists = jnp.sqrt(sq)

        pre1 = Arep + hb + dists * c + feb1[i]
        u = _silu(pre1)
        mij = _silu(_dot(u, few2[i]) + feb2[i]) * mbf

        fxh = _silu(_dot(mij, fxw1[i]) + fxb1[i])
        fx = jnp.sum(fxh * fxw2t[i], axis=1, keepdims=True) + fxb2[i]

        agg = rel * fx
        aggs = agg[0:n]
        mi = mij[0:n]
        for k in range(1, K):
            aggs = aggs + agg[k * n:(k + 1) * n]
            mi = mi + mij[k * n:(k + 1) * n]
        ped = ped + inv_nn * aggs

        fh1 = fhw1[i]
        pre = _dot(h, fh1[0:REPR]) + _dot(mi, fh1[REPR:2 * REPR]) + fhb1[i]
        h = h + _dot(_silu(pre), fhw2[i]) + fhb2[i]

    out_ref[...] = h


@functools.partial(jax.jit, static_argnames=())
def kernel(x, enc_w1, enc_b1, enc_w2, enc_b2, fe_w1, fe_b1, fe_w2, fe_b2,
           fx_w1, fx_b1, fx_w2, fx_b2, fh_w1, fh_b1, fh_w2, fh_b2):
    bs = x.shape[0]
    obs3 = x[:, :HN * OBS].reshape(bs, HN, OBS)
    ped3 = x[:, HN * OBS:].reshape(bs, HN, 2)
    obs2 = obs3.reshape(bs * HN, OBS)
    ped2 = ped3.reshape(bs * HN, 2)
    pedt = jnp.swapaxes(ped3, 1, 2)
    stt = jnp.swapaxes(obs3[..., :STATE], 1, 2)

    eb1 = enc_b1.reshape(1, ENC)
    eb2 = enc_b2.reshape(1, ENC)
    feb1 = fe_b1.reshape(NL, 1, REPR)
    feb2 = fe_b2.reshape(NL, 1, REPR)
    fxb1 = fx_b1.reshape(NL, 1, REPR)
    fxw2t = jnp.swapaxes(fx_w2, 1, 2)
    fxb2 = fx_b2.reshape(NL, 1, 1)
    fhb1 = fh_b1.reshape(NL, 1, REPR)
    fhb2 = fh_b2.reshape(NL, 1, REPR)

    n = SB * HN
    full = lambda arr: pl.BlockSpec(arr.shape, lambda b: (0,) * arr.ndim)
    out = pl.pallas_call(
        _body,
        grid=(bs // SB,),
        in_specs=[
            pl.BlockSpec((n, OBS), lambda b: (b, 0)),
            pl.BlockSpec((n, 2), lambda b: (b, 0)),
            pl.BlockSpec((SB, 2, HN), lambda b: (b, 0, 0)),
            pl.BlockSpec((SB, STATE, HN), lambda b: (b, 0, 0)),
            full(enc_w1), full(eb1), full(enc_w2), full(eb2),
            full(fe_w1), full(feb1), full(fe_w2), full(feb2),
            full(fx_w1), full(fxb1), full(fxw2t), full(fxb2),
            full(fh_w1), full(fhb1), full(fh_w2), full(fhb2),
        ],
        out_specs=pl.BlockSpec((n, REPR), lambda b: (b, 0)),
        out_shape=jax.ShapeDtypeStruct((bs * HN, REPR), jnp.float32),
        compiler_params=pltpu.CompilerParams(dimension_semantics=("parallel",)),
    )(obs2, ped2, pedt, stt,
      enc_w1, eb1, enc_w2, eb2,
      fe_w1, feb1, fe_w2, feb2,
      fx_w1, fxb1, fxw2t, fxb2,
      fh_w1, fhb1, fh_w2, fhb2)
    return out


# speedup vs baseline: 1.1497x; 1.0990x over previous
import functools

import jax
import jax.numpy as jnp
from jax.experimental import pallas as pl
from jax.experimental.pallas import tpu as pltpu

HN = 64
PREF = 16
SCAN = 11
STATE = 6
OBS = SCAN + STATE + PREF
ENC = 128
REPR = ENC + PREF
K = 6
THR = 4.0
NL = 3
SB = 8
BIG = 1e30

_F32 = jnp.float32


def _silu(v):
    return 0.5 * v * (1.0 + jnp.tanh(0.5 * v))


def _dot(a, b):
    return jnp.dot(a, b, preferred_element_type=_F32)


def _body(obs_ref, ped_ref, pedt_ref, stt_ref,
          ew1, eb1, ew2, eb2,
          few1, feb1, few2, feb2,
          fxw1, fxb1, fxw2t, fxb2,
          fhw1, fhb1, fhw2, fhb2,
          out_ref):
    n = SB * HN
    obs = obs_ref[...]
    ped = ped_ref[...]

    ei = obs[:, :STATE + SCAN]
    h1 = jnp.maximum(_dot(ei, ew1[...]) + eb1[...], 0.0)
    henc = _dot(h1, ew2[...]) + eb2[...]
    h = jnp.concatenate([henc, obs[:, STATE + SCAN:OBS]], axis=1)

    stcols = obs[:, :STATE]
    mcol = jnp.max(jnp.where(stcols != 0.0, 1.0, 0.0), axis=1, keepdims=True)
    stt = stt_ref[...]
    mrow3 = jnp.max(jnp.where(stt != 0.0, 1.0, 0.0), axis=1, keepdims=True)
    mrow = jnp.broadcast_to(mrow3, (SB, HN, HN)).reshape(n, HN)

    pedt = pedt_ref[...]
    pxr = jnp.broadcast_to(pedt[:, 0:1, :], (SB, HN, HN)).reshape(n, HN)
    pyr = jnp.broadcast_to(pedt[:, 1:2, :], (SB, HN, HN)).reshape(n, HN)
    dx = pxr - ped[:, 0:1]
    dy = pyr - ped[:, 1:2]
    d2 = dx * dx + dy * dy
    dist = jnp.sqrt(d2)
    valid = (mcol * mrow) > 0.0
    dist = jnp.where(valid, dist, BIG)

    iota = jax.lax.broadcasted_iota(jnp.int32, (n, HN), 1)
    rowid = jax.lax.broadcasted_iota(jnp.int32, (n, 1), 0)
    goff = (rowid // HN) * HN
    giota = jax.lax.broadcasted_iota(jnp.int32, (n, n), 1)
    cur = dist
    w_parts = []
    mb_parts = []
    for _ in range(K):
        mval = jnp.min(cur, axis=1, keepdims=True)
        ismin = cur == mval
        sel = jnp.min(jnp.where(ismin, iota, HN), axis=1, keepdims=True)
        onehot_local = iota == sel
        gsel = sel + goff
        w_parts.append(jnp.where(giota == gsel, 1.0, 0.0))
        mb_parts.append(jnp.where(mval <= THR, 1.0, 0.0))
        cur = jnp.where(onehot_local, BIG, cur)

    W = jnp.concatenate(w_parts, axis=0)
    mbf = jnp.concatenate(mb_parts, axis=0)
    Wm = W * mbf
    neigh_num = mb_parts[0]
    for k in range(1, K):
        neigh_num = neigh_num + mb_parts[k]
    inv_nn = 1.0 / (neigh_num + 1e-6)

    for i in range(NL):
        fe1 = few1[i]
        A = _dot(h, fe1[0:REPR])
        B = _dot(h, fe1[REPR:2 * REPR])
        c = fe1[2 * REPR:2 * REPR + 1]

        hb = _dot(Wm, B)
        Arep = jnp.concatenate([A] * K, axis=0)
        pg = _dot(Wm, ped)
        pedi = jnp.concatenate([ped] * K, axis=0)
        rel = pg - mbf * pedi
        sq = jnp.sum(rel * rel, axis=1, keepdims=True)
        dists = jnp.sqrt(sq)

        pre1 = Arep + hb + dists * c + feb1[i]
        u = _silu(pre1)
        mij = _silu(_dot(u, few2[i]) + feb2[i]) * mbf

        fxh = _silu(_dot(mij, fxw1[i]) + fxb1[i])
        fx = jnp.sum(fxh * fxw2t[i], axis=1, keepdims=True) + fxb2[i]

        agg = rel * fx
        aggs = agg[0:n]
        mi = mij[0:n]
        for k in range(1, K):
            aggs = aggs + agg[k * n:(k + 1) * n]
            mi = mi + mij[k * n:(k + 1) * n]
        ped = ped + inv_nn * aggs

        fh1 = fhw1[i]
        pre = _dot(h, fh1[0:REPR]) + _dot(mi, fh1[REPR:2 * REPR]) + fhb1[i]
        h = h + _dot(_silu(pre), fhw2[i]) + fhb2[i]

    out_ref[...] = h


@functools.partial(jax.jit, static_argnames=())
def kernel(x, enc_w1, enc_b1, enc_w2, enc_b2, fe_w1, fe_b1, fe_w2, fe_b2,
           fx_w1, fx_b1, fx_w2, fx_b2, fh_w1, fh_b1, fh_w2, fh_b2):
    bs = x.shape[0]
    obs3 = x[:, :HN * OBS].reshape(bs, HN, OBS)
    ped3 = x[:, HN * OBS:].reshape(bs, HN, 2)
    obs2 = obs3.reshape(bs * HN, OBS)
    ped2 = ped3.reshape(bs * HN, 2)
    pedt = jnp.swapaxes(ped3, 1, 2)
    stt = jnp.swapaxes(obs3[..., :STATE], 1, 2)

    eb1 = enc_b1.reshape(1, ENC)
    eb2 = enc_b2.reshape(1, ENC)
    feb1 = fe_b1.reshape(NL, 1, REPR)
    feb2 = fe_b2.reshape(NL, 1, REPR)
    fxb1 = fx_b1.reshape(NL, 1, REPR)
    fxw2t = jnp.swapaxes(fx_w2, 1, 2)
    fxb2 = fx_b2.reshape(NL, 1, 1)
    fhb1 = fh_b1.reshape(NL, 1, REPR)
    fhb2 = fh_b2.reshape(NL, 1, REPR)

    n = SB * HN
    full = lambda arr: pl.BlockSpec(arr.shape, lambda b: (0,) * arr.ndim)
    out = pl.pallas_call(
        _body,
        grid=(bs // SB,),
        in_specs=[
            pl.BlockSpec((n, OBS), lambda b: (b, 0)),
            pl.BlockSpec((n, 2), lambda b: (b, 0)),
            pl.BlockSpec((SB, 2, HN), lambda b: (b, 0, 0)),
            pl.BlockSpec((SB, STATE, HN), lambda b: (b, 0, 0)),
            full(enc_w1), full(eb1), full(enc_w2), full(eb2),
            full(fe_w1), full(feb1), full(fe_w2), full(feb2),
            full(fx_w1), full(fxb1), full(fxw2t), full(fxb2),
            full(fh_w1), full(fhb1), full(fh_w2), full(fhb2),
        ],
        out_specs=pl.BlockSpec((n, REPR), lambda b: (b, 0)),
        out_shape=jax.ShapeDtypeStruct((bs * HN, REPR), jnp.float32),
        compiler_params=pltpu.CompilerParams(dimension_semantics=("parallel",)),
    )(obs2, ped2, pedt, stt,
      enc_w1, eb1, enc_w2, eb2,
      fe_w1, feb1, fe_w2, feb2,
      fx_w1, fxb1, fxw2t, fxb2,
      fh_w1, fhb1, fh_w2, fhb2)
    return out
